# Initial kernel scaffold; baseline (speedup 1.0000x reference)
import jax
import jax.numpy as jnp
from jax.experimental import pallas as pl


def kernel(feat, edge_index, W_head, W_tail, W_ent, attn, ln1_g, ln1_b, ln2_g, ln2_b, W1, b1, W2, b2):
    raise NotImplementedError("write your pallas kernel here")


# speedup vs baseline: 1.0076x; 1.0076x over previous
import functools

import jax
import jax.numpy as jnp
from jax.experimental import pallas as pl
from jax.experimental.pallas import tpu as pltpu

N = 10000
E = 160000
D = 256
H = 8
HD = D // H
FF = 4 * D
ALPHA = 0.15
HOPS = 5
SLOPE = 0.2

ROW_BLK = 1000


def _pre_body(feat_ref, wh_ref, wt_ref, we_ref, g_ref, b_ref,
              fh_ref, ft_ref, fe_ref):
    x = feat_ref[:]
    mu = jnp.mean(x, axis=-1, keepdims=True)
    var = jnp.mean((x - mu) ** 2, axis=-1, keepdims=True)
    h = (x - mu) * jax.lax.rsqrt(var + 1e-5) * g_ref[:] + b_ref[:]
    fh_ref[:] = jnp.dot(h, wh_ref[:], preferred_element_type=jnp.float32)
    ft_ref[:] = jnp.dot(h, wt_ref[:], preferred_element_type=jnp.float32)
    fe_ref[:] = jnp.dot(h, we_ref[:], preferred_element_type=jnp.float32)


def _pre_stage(feat, W_head, W_tail, W_ent, ln1_g, ln1_b):
    grid = (N // ROW_BLK,)
    row_spec = pl.BlockSpec((ROW_BLK, D), lambda i: (i, 0))
    full_w = pl.BlockSpec((D, D), lambda i: (0, 0))
    vec_spec = pl.BlockSpec((D,), lambda i: (0,))
    return pl.pallas_call(
        _pre_body,
        grid=grid,
        in_specs=[row_spec, full_w, full_w, full_w, vec_spec, vec_spec],
        out_specs=[row_spec, row_spec, row_spec],
        out_shape=[jax.ShapeDtypeStruct((N, D), jnp.float32)] * 3,
    )(feat, W_head, W_tail, W_ent, ln1_g, ln1_b)


def _post_body(cur_ref, feat_ref, g_ref, b_ref, w1_ref, b1_ref, w2_ref, b2_ref,
               out_ref):
    rst = cur_ref[:] + feat_ref[:]
    mu = jnp.mean(rst, axis=-1, keepdims=True)
    var = jnp.mean((rst - mu) ** 2, axis=-1, keepdims=True)
    h2 = (rst - mu) * jax.lax.rsqrt(var + 1e-5) * g_ref[:] + b_ref[:]
    ff = jnp.maximum(
        jnp.dot(h2, w1_ref[:], preferred_element_type=jnp.float32) + b1_ref[:],
        0.0)
    out_ref[:] = (jnp.dot(ff, w2_ref[:], preferred_element_type=jnp.float32)
                  + b2_ref[:] + rst)


def _post_stage(cur, feat, ln2_g, ln2_b, W1, b1, W2, b2):
    grid = (N // ROW_BLK,)
    row_spec = pl.BlockSpec((ROW_BLK, D), lambda i: (i, 0))
    return pl.pallas_call(
        _post_body,
        grid=grid,
        in_specs=[row_spec, row_spec,
                  pl.BlockSpec((D,), lambda i: (0,)),
                  pl.BlockSpec((D,), lambda i: (0,)),
                  pl.BlockSpec((D, FF), lambda i: (0, 0)),
                  pl.BlockSpec((FF,), lambda i: (0,)),
                  pl.BlockSpec((FF, D), lambda i: (0, 0)),
                  pl.BlockSpec((D,), lambda i: (0,))],
        out_specs=row_spec,
        out_shape=jax.ShapeDtypeStruct((N, D), jnp.float32),
    )(cur, feat, ln2_g, ln2_b, W1, b1, W2, b2)


def kernel(feat, edge_index, W_head, W_tail, W_ent, attn, ln1_g, ln1_b,
           ln2_g, ln2_b, W1, b1, W2, b2):
    src = edge_index[0]
    dst = edge_index[1]
    fh2, ft2, fe2 = _pre_stage(feat, W_head, W_tail, W_ent, ln1_g, ln1_b)
    fh = fh2.reshape(N, H, HD)
    ftail = ft2.reshape(N, H, HD)
    fe = fe2.reshape(N, H, HD)

    e = fh[src] * ftail[dst]
    e = jnp.where(e > 0, e, SLOPE * e)
    e = jnp.sum(e * attn, axis=-1)
    in_deg = jax.ops.segment_sum(jnp.ones((E,), jnp.float32), dst,
                                 num_segments=N)
    log_in = jnp.log(jnp.clip(in_deg, 1.0, None))
    e = e * log_in[dst][:, None] / HD
    ex = jnp.exp(e)
    den = jax.ops.segment_sum(ex, dst, num_segments=N)
    a = ex / jnp.clip(den[dst], 1e-9, None)

    feat0 = fe
    cur = feat0
    for _ in range(HOPS):
        msg = cur[src] * a[:, :, None]
        agg = jax.ops.segment_sum(msg, dst, num_segments=N)
        cur = (1.0 - ALPHA) * agg + ALPHA * feat0

    return _post_stage(cur.reshape(N, D), feat, ln2_g, ln2_b, W1, b1, W2, b2)


# speedup vs baseline: 1.4298x; 1.4191x over previous
import functools

import jax
import jax.numpy as jnp
from jax import lax
from jax.experimental import pallas as pl
from jax.experimental.pallas import tpu as pltpu
from jax.experimental.pallas import tpu_sc as plsc

N = 10000
E = 160000
D = 256
H = 8
HD = D // H
FF = 4 * D
ALPHA = 0.15
HOPS = 5
SLOPE = 0.2

NP = 10240
NP2 = NP // 2
EP = 163840
EPW = EP // 32
CAA = 32
CB = 128
CBB = 512
CH = 128
NRB = NP // 32
ROW_BLK = 1280

_SC_MESH = plsc.VectorSubcoreMesh(core_axis_name="c", subcore_axis_name="s")
_SC_PARAMS = pltpu.CompilerParams(needs_layout_passes=False)


def _pre_body(feat_ref, wh_ref, wt_ref, we_ref, g_ref, b_ref,
              fh_ref, ft_ref, fet_ref):
    x = feat_ref[:]
    mu = jnp.mean(x, axis=-1, keepdims=True)
    var = jnp.mean((x - mu) ** 2, axis=-1, keepdims=True)
    h = (x - mu) * lax.rsqrt(var + 1e-5) * g_ref[:] + b_ref[:]
    fh_ref[:] = jnp.dot(h, wh_ref[:], preferred_element_type=jnp.float32)
    ft_ref[:] = jnp.dot(h, wt_ref[:], preferred_element_type=jnp.float32)
    fe = jnp.dot(h, we_ref[:], preferred_element_type=jnp.float32)
    fet_ref[:] = fe.T.reshape(64, 4, ROW_BLK)


def _pre_stage(featP, W_head, W_tail, W_ent, ln1_g, ln1_b):
    grid = (NP // ROW_BLK,)
    row_spec = pl.BlockSpec((ROW_BLK, D), lambda i: (i, 0))
    full_w = pl.BlockSpec((D, D), lambda i: (0, 0))
    vec_spec = pl.BlockSpec((D,), lambda i: (0,))
    return pl.pallas_call(
        _pre_body,
        grid=grid,
        in_specs=[row_spec, full_w, full_w, full_w, vec_spec, vec_spec],
        out_specs=[row_spec, row_spec,
                   pl.BlockSpec((64, 4, ROW_BLK), lambda i: (0, 0, i))],
        out_shape=[jax.ShapeDtypeStruct((NP, D), jnp.float32),
                   jax.ShapeDtypeStruct((NP, D), jnp.float32),
                   jax.ShapeDtypeStruct((64, 4, NP), jnp.float32)],
    )(featP, W_head, W_tail, W_ent, ln1_g, ln1_b)


def _log_body(dp_ref, scale_ref):
    deg = jnp.sum(dp_ref[:], axis=0)
    scale_ref[:] = jnp.log(jnp.maximum(deg, 1.0)) * (1.0 / HD)


def _log_stage(deg_p):
    return pl.pallas_call(
        _log_body,
        grid=(1,),
        in_specs=[pl.BlockSpec((32, NP), lambda i: (0, 0))],
        out_specs=pl.BlockSpec((NP,), lambda i: (0,)),
        out_shape=jax.ShapeDtypeStruct((NP,), jnp.float32),
    )(deg_p)


def _sc_edge_scores(fh, ft, src_p, dst_p, dst_g, attn_flat):

    @functools.partial(
        pl.kernel,
        out_type=[jax.ShapeDtypeStruct((EP, 16), jnp.float32),
                  jax.ShapeDtypeStruct((32 * NP,), jnp.float32)],
        mesh=_SC_MESH,
        compiler_params=_SC_PARAMS,
        scratch_types=[
            pltpu.VMEM((CAA,), jnp.int32),
            pltpu.VMEM((CAA,), jnp.int32),
            pltpu.VMEM((CAA,), jnp.int32),
            pltpu.VMEM((CAA, D), jnp.float32),
            pltpu.VMEM((CAA, D), jnp.float32),
            pltpu.VMEM((CAA, 16), jnp.float32),
            pltpu.VMEM((D,), jnp.float32),
            pltpu.VMEM((NP,), jnp.float32),
            pltpu.SemaphoreType.DMA,
            pltpu.SemaphoreType.DMA,
        ],
    )
    def body(fh_hbm, ft_hbm, src_hbm, dst_hbm, dstg_hbm, attn_hbm,
             e_hbm, deg_hbm,
             src_c, dst_c, dstg_c, fhr, ftr, ebuf, attn_l, deg_l,
             sem0, sem1):
        c = lax.axis_index("c")
        s = lax.axis_index("s")
        w = s * 2 + c
        lane = lax.iota(jnp.int32, 16)
        one16 = jnp.full((16,), 1.0, jnp.float32)

        pltpu.sync_copy(attn_hbm, attn_l)
        attn_v = [attn_l[pl.ds(k * 16, 16)] for k in range(16)]

        def zdeg(i, _):
            deg_l[pl.ds(i * 16, 16)] = jnp.zeros((16,), jnp.float32)
            return 0
        lax.fori_loop(0, NP // 16, zdeg, 0)

        def chunk(ch, _):
            base = w * EPW + ch * CAA
            pltpu.sync_copy(src_hbm.at[pl.ds(base, CAA)], src_c)
            pltpu.sync_copy(dst_hbm.at[pl.ds(base, CAA)], dst_c)
            pltpu.sync_copy(dstg_hbm.at[pl.ds(base, CAA)], dstg_c)
            cp0 = pltpu.async_copy(fh_hbm.at[src_c], fhr, sem0)
            cp1 = pltpu.async_copy(ft_hbm.at[dstg_c], ftr, sem1)
            cp0.wait()
            cp1.wait()

            def edge(i, _):
                tots = []
                for h in range(H):
                    acc = jnp.zeros((16,), jnp.float32)
                    for k in (2 * h, 2 * h + 1):
                        a_ = fhr[i, pl.ds(k * 16, 16)]
                        b_ = ftr[i, pl.ds(k * 16, 16)]
                        p = a_ * b_
                        p = jnp.where(p > 0, p, SLOPE * p)
                        acc = acc + p * attn_v[k]
                    tots.append(jnp.sum(acc))
                row = jnp.zeros((16,), jnp.float32)
                for h in range(H):
                    row = jnp.where(lane == h, tots[h], row)
                ebuf[i, :] = row
                return 0
            lax.fori_loop(0, CAA, edge, 0)

            for g in range(CAA // 16):
                dstv = dst_c[pl.ds(g * 16, 16)]
                plsc.addupdate_scatter(deg_l, [dstv], one16)

            pltpu.sync_copy(ebuf, e_hbm.at[pl.ds(base, CAA)])
            return 0
        lax.fori_loop(0, EPW // CAA, chunk, 0)

        pltpu.sync_copy(deg_l, deg_hbm.at[pl.ds(w * NP, NP)])

    return body(fh, ft, src_p, dst_p, dst_g, attn_flat)


def _sc_den(e_raw, dst_p, scale):

    @functools.partial(
        pl.kernel,
        out_type=jax.ShapeDtypeStruct((32, NRB, 128), jnp.float32),
        mesh=_SC_MESH,
        compiler_params=_SC_PARAMS,
        scratch_types=[
            pltpu.VMEM((CBB,), jnp.int32),
            pltpu.VMEM((CBB, 16), jnp.float32),
            pltpu.VMEM((NRB, 128), jnp.float32),
            pltpu.VMEM((NP,), jnp.float32),
        ],
    )
    def body(e_hbm, dst_hbm, scale_hbm, den_hbm,
             dst_c, ec, den_l, scale_l):
        c = lax.axis_index("c")
        s = lax.axis_index("s")
        w = s * 2 + c
        lane = lax.iota(jnp.int32, 16)
        lo = w * NRB

        pltpu.sync_copy(scale_hbm, scale_l)

        def zd(i, _):
            for kk in range(8):
                den_l[i, pl.ds(kk * 16, 16)] = jnp.zeros((16,), jnp.float32)
            return 0
        lax.fori_loop(0, NRB, zd, 0)

        def chunk(ch, _):
            base = ch * CBB
            pltpu.sync_copy(dst_hbm.at[pl.ds(base, CBB)], dst_c)
            pltpu.sync_copy(e_hbm.at[pl.ds(base, CBB)], ec)

            def grp(g, _):
                r = g * 16 + lane
                dv = dst_c[pl.ds(g * 16, 16)]
                sv = plsc.load_gather(scale_l, [dv])
                m = (dv >= lo) & (dv < lo + NRB)
                dloc = jnp.clip(dv - lo, 0, NRB - 1)
                for h in range(H):
                    hc = jnp.full((16,), h, jnp.int32)
                    ev = plsc.load_gather(ec, [r, hc])
                    ex = jnp.exp(ev * sv)
                    plsc.addupdate_scatter(den_l, [dloc, hc], ex, mask=m)
                return 0
            lax.fori_loop(0, CBB // 16, grp, 0)
            return 0
        lax.fori_loop(0, EP // CBB, chunk, 0)

        pltpu.sync_copy(den_l, den_hbm.at[w])

    return body(e_raw, dst_p, scale)


def _sc_attn(e_raw, dst_p, scale, den2):

    @functools.partial(
        pl.kernel,
        out_type=jax.ShapeDtypeStruct((2 * EP, 4), jnp.float32),
        mesh=_SC_MESH,
        compiler_params=_SC_PARAMS,
        scratch_types=[
            pltpu.VMEM((CB,), jnp.int32),
            pltpu.VMEM((CB, 16), jnp.float32),
            pltpu.VMEM((CB, 128), jnp.float32),
            pltpu.VMEM((CB, 4), jnp.float32),
            pltpu.VMEM((CB, 4), jnp.float32),
            pltpu.VMEM((NP,), jnp.float32),
            pltpu.SemaphoreType.DMA,
        ],
    )
    def body(e_hbm, dst_hbm, scale_hbm, den_hbm, a_hbm,
             dst_c, ec, denr, ab0, ab1, scale_l, sem0):
        c = lax.axis_index("c")
        s = lax.axis_index("s")
        w = s * 2 + c
        lane = lax.iota(jnp.int32, 16)

        pltpu.sync_copy(scale_hbm, scale_l)

        def chunk(ch, _):
            base = w * EPW + ch * CB
            pltpu.sync_copy(dst_hbm.at[pl.ds(base, CB)], dst_c)
            pltpu.sync_copy(e_hbm.at[pl.ds(base, CB)], ec)
            pltpu.async_copy(den_hbm.at[dst_c], denr, sem0).wait()

            def grp(g, _):
                r = g * 16 + lane
                dv = dst_c[pl.ds(g * 16, 16)]
                sv = plsc.load_gather(scale_l, [dv])
                for h in range(H):
                    hc = jnp.full((16,), h, jnp.int32)
                    ev = plsc.load_gather(ec, [r, hc])
                    ex = jnp.exp(ev * sv)
                    dnv = plsc.load_gather(denr, [r, hc])
                    av = ex / dnv
                    hc4 = jnp.full((16,), h % 4, jnp.int32)
                    if h < 4:
                        plsc.store_scatter(ab0, [r, hc4], av)
                    else:
                        plsc.store_scatter(ab1, [r, hc4], av)
                return 0
            lax.fori_loop(0, CB // 16, grp, 0)
            pltpu.sync_copy(ab0, a_hbm.at[pl.ds(base, CB)])
            pltpu.sync_copy(ab1, a_hbm.at[pl.ds(EP + base, CB)])
            return 0
        lax.fori_loop(0, EPW // CB, chunk, 0)

    return body(e_raw, dst_p, scale, den2)


def _sc_hop(curA, feT, src_p, dst_p, a2):

    @functools.partial(
        pl.kernel,
        out_type=jax.ShapeDtypeStruct((64, 4, NP), jnp.float32),
        mesh=_SC_MESH,
        compiler_params=_SC_PARAMS,
        scratch_types=[
            pltpu.VMEM((CH,), jnp.int32),
            pltpu.VMEM((CH,), jnp.int32),
            pltpu.VMEM((CH, 4), jnp.float32),
            pltpu.VMEM((4, NP), jnp.float32),
            pltpu.VMEM((4, NP2), jnp.float32),
            pltpu.VMEM((4, 512), jnp.float32),
        ],
    )
    def body(cur_hbm, fet_hbm, src_hbm, dst_hbm, a_hbm, out_hbm,
             src_c, dst_c, ac, curl, agg, fel):
        c = lax.axis_index("c")
        s = lax.axis_index("s")
        w = s * 2 + c
        lane = lax.iota(jnp.int32, 16)

        for gi in range(2):
            g4 = w * 2 + gi
            c4 = g4 // 32
            lh = (g4 // 8) % 4
            pltpu.sync_copy(cur_hbm.at[g4], curl)

            for nh in range(2):
                lo = nh * NP2

                for j in range(4):
                    def za(i, _, j=j):
                        agg[j, pl.ds(i * 16, 16)] = jnp.zeros((16,),
                                                              jnp.float32)
                        return 0
                    lax.fori_loop(0, NP2 // 16, za, 0)

                def chunk(ch, _):
                    base = ch * CH
                    pltpu.sync_copy(src_hbm.at[pl.ds(base, CH)], src_c)
                    pltpu.sync_copy(dst_hbm.at[pl.ds(base, CH)], dst_c)
                    pltpu.sync_copy(a_hbm.at[pl.ds(c4 * EP + base, CH)], ac)

                    def grp(g, _):
                        r = g * 16 + lane
                        sv = src_c[pl.ds(g * 16, 16)]
                        dv = dst_c[pl.ds(g * 16, 16)]
                        lhc = jnp.full((16,), lh, jnp.int32)
                        av = plsc.load_gather(ac, [r, lhc])
                        m = (dv >= lo) & (dv < lo + NP2)
                        dloc = jnp.clip(dv - lo, 0, NP2 - 1)
                        for j in range(4):
                            jc = jnp.full((16,), j, jnp.int32)
                            cv = plsc.load_gather(curl, [jc, sv])
                            plsc.addupdate_scatter(agg, [jc, dloc],
                                                   cv * av, mask=m)
                        return 0
                    lax.fori_loop(0, CH // 16, grp, 0)
                    return 0
                lax.fori_loop(0, EP // CH, chunk, 0)

                for off in range(0, NP2, 512):
                    pltpu.sync_copy(
                        fet_hbm.at[g4, :, pl.ds(lo + off, 512)], fel)
                    for j in range(4):
                        def bl(t, _, j=j, off=off):
                            slq = pl.ds(off + t * 16, 16)
                            slf = pl.ds(t * 16, 16)
                            agg[j, slq] = ((1.0 - ALPHA) * agg[j, slq]
                                           + ALPHA * fel[j, slf])
                            return 0
                        lax.fori_loop(0, 512 // 16, bl, 0)
                pltpu.sync_copy(agg, out_hbm.at[g4, :, pl.ds(lo, NP2)])

    return body(curA, feT, src_p, dst_p, a2)


def _post_body(ct_ref, feat_ref, g_ref, b_ref, w1_ref, b1_ref,
               w2_ref, b2_ref, out_ref):
    cur = ct_ref[:].reshape(D, ROW_BLK).T
    rst = cur + feat_ref[:]
    mu = jnp.mean(rst, axis=-1, keepdims=True)
    var = jnp.mean((rst - mu) ** 2, axis=-1, keepdims=True)
    h2 = (rst - mu) * lax.rsqrt(var + 1e-5) * g_ref[:] + b_ref[:]
    ff = jnp.maximum(
        jnp.dot(h2, w1_ref[:], preferred_element_type=jnp.float32) + b1_ref[:],
        0.0)
    out_ref[:] = (jnp.dot(ff, w2_ref[:], preferred_element_type=jnp.float32)
                  + b2_ref[:] + rst)


def _post_stage(curT, featP, ln2_g, ln2_b, W1, b1, W2, b2):
    grid = (NP // ROW_BLK,)
    row_spec = pl.BlockSpec((ROW_BLK, D), lambda i: (i, 0))
    return pl.pallas_call(
        _post_body,
        grid=grid,
        in_specs=[pl.BlockSpec((64, 4, ROW_BLK), lambda i: (0, 0, i)),
                  row_spec,
                  pl.BlockSpec((D,), lambda i: (0,)),
                  pl.BlockSpec((D,), lambda i: (0,)),
                  pl.BlockSpec((D, FF), lambda i: (0, 0)),
                  pl.BlockSpec((FF,), lambda i: (0,)),
                  pl.BlockSpec((FF, D), lambda i: (0, 0)),
                  pl.BlockSpec((D,), lambda i: (0,))],
        out_specs=row_spec,
        out_shape=jax.ShapeDtypeStruct((NP, D), jnp.float32),
    )(curT, featP, ln2_g, ln2_b, W1, b1, W2, b2)


def kernel(feat, edge_index, W_head, W_tail, W_ent, attn, ln1_g, ln1_b,
           ln2_g, ln2_b, W1, b1, W2, b2):
    src_p = jnp.concatenate(
        [edge_index[0], jnp.zeros((EP - E,), jnp.int32)])
    dst_p = jnp.concatenate(
        [edge_index[1], jnp.full((EP - E,), N, jnp.int32)])
    dst_g = jnp.concatenate(
        [edge_index[1], jnp.zeros((EP - E,), jnp.int32)])
    attn_flat = attn.reshape(D)
    featP = jnp.concatenate(
        [feat, jnp.zeros((NP - N, D), jnp.float32)], axis=0)

    fh, ft, feT = _pre_stage(featP, W_head, W_tail, W_ent, ln1_g, ln1_b)

    e_raw, deg = _sc_edge_scores(fh, ft, src_p, dst_p, dst_g, attn_flat)
    scale = _log_stage(deg.reshape(32, NP))
    den3 = _sc_den(e_raw, dst_p, scale)
    a2 = _sc_attn(e_raw, dst_p, scale, den3.reshape(NP, 128))

    cur = feT
    for _ in range(HOPS):
        cur = _sc_hop(cur, feT, src_p, dst_p, a2)

    out = _post_stage(cur, featP, ln2_g, ln2_b, W1, b1, W2, b2)
    return out[:N]


# speedup vs baseline: 2.5265x; 1.7670x over previous
import functools

import jax
import jax.numpy as jnp
from jax import lax
from jax.experimental import pallas as pl
from jax.experimental.pallas import tpu as pltpu
from jax.experimental.pallas import tpu_sc as plsc

N = 10000
E = 160000
D = 256
H = 8
HD = D // H
FF = 4 * D
ALPHA = 0.15
HOPS = 5
SLOPE = 0.2

NP = 10240
NP2 = NP // 2
EP = 163840
EPW = EP // 32
CAA = 64
CB = 128
CBB = 512
CH = 512
NRB = NP // 32
ROW_BLK = 1280

_SC_MESH = plsc.VectorSubcoreMesh(core_axis_name="c", subcore_axis_name="s")
_SC_PARAMS = pltpu.CompilerParams(needs_layout_passes=False)


def _pre_body(feat_ref, wh_ref, wt_ref, we_ref, g_ref, b_ref,
              fh_ref, ft_ref, fet_ref):
    x = feat_ref[:]
    mu = jnp.mean(x, axis=-1, keepdims=True)
    var = jnp.mean((x - mu) ** 2, axis=-1, keepdims=True)
    h = (x - mu) * lax.rsqrt(var + 1e-5) * g_ref[:] + b_ref[:]
    fh_ref[:] = jnp.dot(h, wh_ref[:], preferred_element_type=jnp.float32)
    ft_ref[:] = jnp.dot(h, wt_ref[:], preferred_element_type=jnp.float32)
    fe = jnp.dot(h, we_ref[:], preferred_element_type=jnp.float32)
    fet_ref[:] = fe.T.reshape(64, 4, ROW_BLK)


def _pre_stage(featP, W_head, W_tail, W_ent, ln1_g, ln1_b):
    grid = (NP // ROW_BLK,)
    row_spec = pl.BlockSpec((ROW_BLK, D), lambda i: (i, 0))
    full_w = pl.BlockSpec((D, D), lambda i: (0, 0))
    vec_spec = pl.BlockSpec((D,), lambda i: (0,))
    return pl.pallas_call(
        _pre_body,
        grid=grid,
        in_specs=[row_spec, full_w, full_w, full_w, vec_spec, vec_spec],
        out_specs=[row_spec, row_spec,
                   pl.BlockSpec((64, 4, ROW_BLK), lambda i: (0, 0, i))],
        out_shape=[jax.ShapeDtypeStruct((NP, D), jnp.float32),
                   jax.ShapeDtypeStruct((NP, D), jnp.float32),
                   jax.ShapeDtypeStruct((64, 4, NP), jnp.float32)],
    )(featP, W_head, W_tail, W_ent, ln1_g, ln1_b)


def _log_body(dp_ref, scale_ref):
    deg = jnp.sum(dp_ref[:], axis=0)
    scale_ref[:] = jnp.log(jnp.maximum(deg, 1.0)) * (1.0 / HD)


def _log_stage(deg_p):
    return pl.pallas_call(
        _log_body,
        grid=(1,),
        in_specs=[pl.BlockSpec((32, NP), lambda i: (0, 0))],
        out_specs=pl.BlockSpec((NP,), lambda i: (0,)),
        out_shape=jax.ShapeDtypeStruct((NP,), jnp.float32),
    )(deg_p)


def _sc_edge_scores(fh, ft, src_p, dst_p, dst_g, attn_flat):

    @functools.partial(
        pl.kernel,
        out_type=[jax.ShapeDtypeStruct((EP, 16), jnp.float32),
                  jax.ShapeDtypeStruct((32 * NP,), jnp.float32)],
        mesh=_SC_MESH,
        compiler_params=_SC_PARAMS,
        scratch_types=[
            pltpu.VMEM((CAA,), jnp.int32),
            pltpu.VMEM((CAA,), jnp.int32),
            pltpu.VMEM((CAA,), jnp.int32),
            pltpu.VMEM((CAA, D), jnp.float32),
            pltpu.VMEM((CAA, D), jnp.float32),
            pltpu.VMEM((CAA, 16), jnp.float32),
            pltpu.VMEM((D,), jnp.float32),
            pltpu.VMEM((NP,), jnp.float32),
            pltpu.SemaphoreType.DMA,
            pltpu.SemaphoreType.DMA,
        ],
    )
    def body(fh_hbm, ft_hbm, src_hbm, dst_hbm, dstg_hbm, attn_hbm,
             e_hbm, deg_hbm,
             src_c, dst_c, dstg_c, fhr, ftr, ebuf, attn_l, deg_l,
             sem0, sem1):
        c = lax.axis_index("c")
        s = lax.axis_index("s")
        w = s * 2 + c
        lane = lax.iota(jnp.int32, 16)
        one16 = jnp.full((16,), 1.0, jnp.float32)

        pltpu.sync_copy(attn_hbm, attn_l)
        attn_v = [attn_l[pl.ds(k * 16, 16)] for k in range(16)]

        def zdeg(i, _):
            deg_l[pl.ds(i * 16, 16)] = jnp.zeros((16,), jnp.float32)
            return 0
        lax.fori_loop(0, NP // 16, zdeg, 0)

        def chunk(ch, _):
            base = w * EPW + ch * CAA
            pltpu.sync_copy(src_hbm.at[pl.ds(base, CAA)], src_c)
            pltpu.sync_copy(dst_hbm.at[pl.ds(base, CAA)], dst_c)
            pltpu.sync_copy(dstg_hbm.at[pl.ds(base, CAA)], dstg_c)
            cp0 = pltpu.async_copy(fh_hbm.at[src_c], fhr, sem0)
            cp1 = pltpu.async_copy(ft_hbm.at[dstg_c], ftr, sem1)
            cp0.wait()
            cp1.wait()

            def edge(i, _):
                tots = []
                for h in range(H):
                    acc = jnp.zeros((16,), jnp.float32)
                    for k in (2 * h, 2 * h + 1):
                        a_ = fhr[i, pl.ds(k * 16, 16)]
                        b_ = ftr[i, pl.ds(k * 16, 16)]
                        p = a_ * b_
                        p = jnp.where(p > 0, p, SLOPE * p)
                        acc = acc + p * attn_v[k]
                    tots.append(jnp.sum(acc))
                row = jnp.zeros((16,), jnp.float32)
                for h in range(H):
                    row = jnp.where(lane == h, tots[h], row)
                ebuf[i, :] = row
                return 0
            lax.fori_loop(0, CAA, edge, 0)

            for g in range(CAA // 16):
                dstv = dst_c[pl.ds(g * 16, 16)]
                plsc.addupdate_scatter(deg_l, [dstv], one16)

            pltpu.sync_copy(ebuf, e_hbm.at[pl.ds(base, CAA)])
            return 0
        lax.fori_loop(0, EPW // CAA, chunk, 0)

        pltpu.sync_copy(deg_l, deg_hbm.at[pl.ds(w * NP, NP)])

    return body(fh, ft, src_p, dst_p, dst_g, attn_flat)


def _sc_den(e_raw, dst_p, scale):

    @functools.partial(
        pl.kernel,
        out_type=jax.ShapeDtypeStruct((32, NRB, 128), jnp.float32),
        mesh=_SC_MESH,
        compiler_params=_SC_PARAMS,
        scratch_types=[
            pltpu.VMEM((CBB,), jnp.int32),
            pltpu.VMEM((CBB, 16), jnp.float32),
            pltpu.VMEM((NRB, 128), jnp.float32),
            pltpu.VMEM((NP,), jnp.float32),
        ],
    )
    def body(e_hbm, dst_hbm, scale_hbm, den_hbm,
             dst_c, ec, den_l, scale_l):
        c = lax.axis_index("c")
        s = lax.axis_index("s")
        w = s * 2 + c
        lane = lax.iota(jnp.int32, 16)
        lo = w * NRB

        pltpu.sync_copy(scale_hbm, scale_l)

        def zd(i, _):
            for kk in range(8):
                den_l[i, pl.ds(kk * 16, 16)] = jnp.zeros((16,), jnp.float32)
            return 0
        lax.fori_loop(0, NRB, zd, 0)

        def chunk(ch, _):
            base = ch * CBB
            pltpu.sync_copy(dst_hbm.at[pl.ds(base, CBB)], dst_c)
            pltpu.sync_copy(e_hbm.at[pl.ds(base, CBB)], ec)

            def grp(g, _):
                r = g * 16 + lane
                dv = dst_c[pl.ds(g * 16, 16)]
                sv = plsc.load_gather(scale_l, [dv])
                m = (dv >= lo) & (dv < lo + NRB)
                dloc = jnp.clip(dv - lo, 0, NRB - 1)
                for h in range(H):
                    hc = jnp.full((16,), h, jnp.int32)
                    ev = plsc.load_gather(ec, [r, hc])
                    ex = jnp.exp(ev * sv)
                    plsc.addupdate_scatter(den_l, [dloc, hc], ex, mask=m)
                return 0
            lax.fori_loop(0, CBB // 16, grp, 0)
            return 0
        lax.fori_loop(0, EP // CBB, chunk, 0)

        pltpu.sync_copy(den_l, den_hbm.at[w])

    return body(e_raw, dst_p, scale)


def _sc_attn(e_raw, dst_p, scale, den2):

    @functools.partial(
        pl.kernel,
        out_type=jax.ShapeDtypeStruct((2 * EP, 4), jnp.float32),
        mesh=_SC_MESH,
        compiler_params=_SC_PARAMS,
        scratch_types=[
            pltpu.VMEM((CB,), jnp.int32),
            pltpu.VMEM((CB, 16), jnp.float32),
            pltpu.VMEM((CB, 128), jnp.float32),
            pltpu.VMEM((CB, 4), jnp.float32),
            pltpu.VMEM((CB, 4), jnp.float32),
            pltpu.VMEM((NP,), jnp.float32),
            pltpu.SemaphoreType.DMA,
        ],
    )
    def body(e_hbm, dst_hbm, scale_hbm, den_hbm, a_hbm,
             dst_c, ec, denr, ab0, ab1, scale_l, sem0):
        c = lax.axis_index("c")
        s = lax.axis_index("s")
        w = s * 2 + c
        lane = lax.iota(jnp.int32, 16)

        pltpu.sync_copy(scale_hbm, scale_l)

        def chunk(ch, _):
            base = w * EPW + ch * CB
            pltpu.sync_copy(dst_hbm.at[pl.ds(base, CB)], dst_c)
            pltpu.sync_copy(e_hbm.at[pl.ds(base, CB)], ec)
            pltpu.async_copy(den_hbm.at[dst_c], denr, sem0).wait()

            def grp(g, _):
                r = g * 16 + lane
                dv = dst_c[pl.ds(g * 16, 16)]
                sv = plsc.load_gather(scale_l, [dv])
                for h in range(H):
                    hc = jnp.full((16,), h, jnp.int32)
                    ev = plsc.load_gather(ec, [r, hc])
                    ex = jnp.exp(ev * sv)
                    dnv = plsc.load_gather(denr, [r, hc])
                    av = ex / dnv
                    hc4 = jnp.full((16,), h % 4, jnp.int32)
                    if h < 4:
                        plsc.store_scatter(ab0, [r, hc4], av)
                    else:
                        plsc.store_scatter(ab1, [r, hc4], av)
                return 0
            lax.fori_loop(0, CB // 16, grp, 0)
            pltpu.sync_copy(ab0, a_hbm.at[pl.ds(base, CB)])
            pltpu.sync_copy(ab1, a_hbm.at[pl.ds(EP + base, CB)])
            return 0
        lax.fori_loop(0, EPW // CB, chunk, 0)

    return body(e_raw, dst_p, scale, den2)


def _sc_hop(curA, feT, src_p, dst_p, a2):

    @functools.partial(
        pl.kernel,
        out_type=jax.ShapeDtypeStruct((64, 4, NP), jnp.float32),
        mesh=_SC_MESH,
        compiler_params=_SC_PARAMS,
        scratch_types=[
            pltpu.VMEM((CH,), jnp.int32),
            pltpu.VMEM((CH,), jnp.int32),
            pltpu.VMEM((CH, 4), jnp.float32),
            pltpu.VMEM((4, NP), jnp.float32),
            pltpu.VMEM((4, NP2), jnp.float32),
            pltpu.VMEM((4, 128), jnp.float32),
        ],
    )
    def body(cur_hbm, fet_hbm, src_hbm, dst_hbm, a_hbm, out_hbm,
             src_c, dst_c, ac, curl, agg, fel):
        c = lax.axis_index("c")
        s = lax.axis_index("s")
        w = s * 2 + c
        lane = lax.iota(jnp.int32, 16)

        for gi in range(2):
            g4 = w * 2 + gi
            c4 = g4 // 32
            lh = (g4 // 8) % 4
            pltpu.sync_copy(cur_hbm.at[g4], curl)

            for nh in range(2):
                lo = nh * NP2

                for j in range(4):
                    def za(i, _, j=j):
                        agg[j, pl.ds(i * 16, 16)] = jnp.zeros((16,),
                                                              jnp.float32)
                        return 0
                    lax.fori_loop(0, NP2 // 16, za, 0)

                def chunk(ch, _):
                    base = ch * CH
                    pltpu.sync_copy(src_hbm.at[pl.ds(base, CH)], src_c)
                    pltpu.sync_copy(dst_hbm.at[pl.ds(base, CH)], dst_c)
                    pltpu.sync_copy(a_hbm.at[pl.ds(c4 * EP + base, CH)], ac)

                    def grp(g, _):
                        r = g * 16 + lane
                        sv = src_c[pl.ds(g * 16, 16)]
                        dv = dst_c[pl.ds(g * 16, 16)]
                        lhc = jnp.full((16,), lh, jnp.int32)
                        av = plsc.load_gather(ac, [r, lhc])
                        m = (dv >= lo) & (dv < lo + NP2)
                        dloc = jnp.clip(dv - lo, 0, NP2 - 1)
                        for j in range(4):
                            jc = jnp.full((16,), j, jnp.int32)
                            cv = plsc.load_gather(curl, [jc, sv])
                            plsc.addupdate_scatter(agg, [jc, dloc],
                                                   cv * av, mask=m)
                        return 0
                    lax.fori_loop(0, CH // 16, grp, 0)
                    return 0
                lax.fori_loop(0, EP // CH, chunk, 0)

                def blch(cc, _):
                    off = cc * 128
                    pltpu.sync_copy(
                        fet_hbm.at[g4, :, pl.ds(lo + off, 128)], fel)
                    for j in range(4):
                        def bl(t, _, j=j):
                            slq = pl.ds(off + t * 16, 16)
                            slf = pl.ds(t * 16, 16)
                            agg[j, slq] = ((1.0 - ALPHA) * agg[j, slq]
                                           + ALPHA * fel[j, slf])
                            return 0
                        lax.fori_loop(0, 128 // 16, bl, 0)
                    return 0
                lax.fori_loop(0, NP2 // 128, blch, 0)
                pltpu.sync_copy(agg, out_hbm.at[g4, :, pl.ds(lo, NP2)])

    return body(curA, feT, src_p, dst_p, a2)


def _post_body(ct_ref, feat_ref, g_ref, b_ref, w1_ref, b1_ref,
               w2_ref, b2_ref, out_ref):
    cur = ct_ref[:].reshape(D, ROW_BLK).T
    rst = cur + feat_ref[:]
    mu = jnp.mean(rst, axis=-1, keepdims=True)
    var = jnp.mean((rst - mu) ** 2, axis=-1, keepdims=True)
    h2 = (rst - mu) * lax.rsqrt(var + 1e-5) * g_ref[:] + b_ref[:]
    ff = jnp.maximum(
        jnp.dot(h2, w1_ref[:], preferred_element_type=jnp.float32) + b1_ref[:],
        0.0)
    out_ref[:] = (jnp.dot(ff, w2_ref[:], preferred_element_type=jnp.float32)
                  + b2_ref[:] + rst)


def _post_stage(curT, featP, ln2_g, ln2_b, W1, b1, W2, b2):
    grid = (NP // ROW_BLK,)
    row_spec = pl.BlockSpec((ROW_BLK, D), lambda i: (i, 0))
    return pl.pallas_call(
        _post_body,
        grid=grid,
        in_specs=[pl.BlockSpec((64, 4, ROW_BLK), lambda i: (0, 0, i)),
                  row_spec,
                  pl.BlockSpec((D,), lambda i: (0,)),
                  pl.BlockSpec((D,), lambda i: (0,)),
                  pl.BlockSpec((D, FF), lambda i: (0, 0)),
                  pl.BlockSpec((FF,), lambda i: (0,)),
                  pl.BlockSpec((FF, D), lambda i: (0, 0)),
                  pl.BlockSpec((D,), lambda i: (0,))],
        out_specs=row_spec,
        out_shape=jax.ShapeDtypeStruct((NP, D), jnp.float32),
    )(curT, featP, ln2_g, ln2_b, W1, b1, W2, b2)


def kernel(feat, edge_index, W_head, W_tail, W_ent, attn, ln1_g, ln1_b,
           ln2_g, ln2_b, W1, b1, W2, b2):
    src_p = jnp.concatenate(
        [edge_index[0], jnp.zeros((EP - E,), jnp.int32)])
    dst_p = jnp.concatenate(
        [edge_index[1], jnp.full((EP - E,), N, jnp.int32)])
    dst_g = jnp.concatenate(
        [edge_index[1], jnp.zeros((EP - E,), jnp.int32)])
    attn_flat = attn.reshape(D)
    featP = jnp.concatenate(
        [feat, jnp.zeros((NP - N, D), jnp.float32)], axis=0)

    fh, ft, feT = _pre_stage(featP, W_head, W_tail, W_ent, ln1_g, ln1_b)

    e_raw, deg = _sc_edge_scores(fh, ft, src_p, dst_p, dst_g, attn_flat)
    scale = _log_stage(deg.reshape(32, NP))
    den3 = _sc_den(e_raw, dst_p, scale)
    a2 = _sc_attn(e_raw, dst_p, scale, den3.reshape(NP, 128))

    cur = feT
    for _ in range(HOPS):
        cur = _sc_hop(cur, feT, src_p, dst_p, a2)

    out = _post_stage(cur, featP, ln2_g, ln2_b, W1, b1, W2, b2)
    return out[:N]


# speedup vs baseline: 2.5705x; 1.0174x over previous
import functools

import jax
import jax.numpy as jnp
from jax import lax
from jax.experimental import pallas as pl
from jax.experimental.pallas import tpu as pltpu
from jax.experimental.pallas import tpu_sc as plsc

N = 10000
E = 160000
D = 256
H = 8
HD = D // H
FF = 4 * D
ALPHA = 0.15
HOPS = 5
SLOPE = 0.2

NP = 10240
NP2 = NP // 2
EP = 163840
EPW = EP // 32
CAA = 64
CB = 128
CBB = 512
CH = 512
NRB = NP // 32
ROW_BLK = 1280

_SC_MESH = plsc.VectorSubcoreMesh(core_axis_name="c", subcore_axis_name="s")
_SC_PARAMS = pltpu.CompilerParams(needs_layout_passes=False)


def _pre_body(feat_ref, wh_ref, wt_ref, we_ref, g_ref, b_ref,
              fh_ref, ft_ref, fet_ref):
    x = feat_ref[:]
    mu = jnp.mean(x, axis=-1, keepdims=True)
    var = jnp.mean((x - mu) ** 2, axis=-1, keepdims=True)
    h = (x - mu) * lax.rsqrt(var + 1e-5) * g_ref[:] + b_ref[:]
    fh_ref[:] = jnp.dot(h, wh_ref[:], preferred_element_type=jnp.float32)
    ft_ref[:] = jnp.dot(h, wt_ref[:], preferred_element_type=jnp.float32)
    fe = jnp.dot(h, we_ref[:], preferred_element_type=jnp.float32)
    fet_ref[:] = fe.T.reshape(128, 2, ROW_BLK)


def _pre_stage(featP, W_head, W_tail, W_ent, ln1_g, ln1_b):
    grid = (NP // ROW_BLK,)
    row_spec = pl.BlockSpec((ROW_BLK, D), lambda i: (i, 0))
    full_w = pl.BlockSpec((D, D), lambda i: (0, 0))
    vec_spec = pl.BlockSpec((D,), lambda i: (0,))
    return pl.pallas_call(
        _pre_body,
        grid=grid,
        in_specs=[row_spec, full_w, full_w, full_w, vec_spec, vec_spec],
        out_specs=[row_spec, row_spec,
                   pl.BlockSpec((128, 2, ROW_BLK), lambda i: (0, 0, i))],
        out_shape=[jax.ShapeDtypeStruct((NP, D), jnp.float32),
                   jax.ShapeDtypeStruct((NP, D), jnp.float32),
                   jax.ShapeDtypeStruct((128, 2, NP), jnp.float32)],
    )(featP, W_head, W_tail, W_ent, ln1_g, ln1_b)


def _log_body(dp_ref, scale_ref):
    deg = jnp.sum(dp_ref[:], axis=0)
    scale_ref[:] = jnp.log(jnp.maximum(deg, 1.0)) * (1.0 / HD)


def _log_stage(deg_p):
    return pl.pallas_call(
        _log_body,
        grid=(1,),
        in_specs=[pl.BlockSpec((32, NP), lambda i: (0, 0))],
        out_specs=pl.BlockSpec((NP,), lambda i: (0,)),
        out_shape=jax.ShapeDtypeStruct((NP,), jnp.float32),
    )(deg_p)


def _sc_edge_scores(fh, ft, src_p, dst_p, dst_g, attn_flat):

    @functools.partial(
        pl.kernel,
        out_type=[jax.ShapeDtypeStruct((EP, 16), jnp.float32),
                  jax.ShapeDtypeStruct((32 * NP,), jnp.float32)],
        mesh=_SC_MESH,
        compiler_params=_SC_PARAMS,
        scratch_types=[
            pltpu.VMEM((CAA,), jnp.int32),
            pltpu.VMEM((CAA,), jnp.int32),
            pltpu.VMEM((CAA,), jnp.int32),
            pltpu.VMEM((CAA, D), jnp.float32),
            pltpu.VMEM((CAA, D), jnp.float32),
            pltpu.VMEM((CAA, 16), jnp.float32),
            pltpu.VMEM((D,), jnp.float32),
            pltpu.VMEM((NP,), jnp.float32),
            pltpu.SemaphoreType.DMA,
            pltpu.SemaphoreType.DMA,
        ],
    )
    def body(fh_hbm, ft_hbm, src_hbm, dst_hbm, dstg_hbm, attn_hbm,
             e_hbm, deg_hbm,
             src_c, dst_c, dstg_c, fhr, ftr, ebuf, attn_l, deg_l,
             sem0, sem1):
        c = lax.axis_index("c")
        s = lax.axis_index("s")
        w = s * 2 + c
        lane = lax.iota(jnp.int32, 16)
        one16 = jnp.full((16,), 1.0, jnp.float32)

        pltpu.sync_copy(attn_hbm, attn_l)
        attn_v = [attn_l[pl.ds(k * 16, 16)] for k in range(16)]

        def zdeg(i, _):
            deg_l[pl.ds(i * 16, 16)] = jnp.zeros((16,), jnp.float32)
            return 0
        lax.fori_loop(0, NP // 16, zdeg, 0)

        def chunk(ch, _):
            base = w * EPW + ch * CAA
            pltpu.sync_copy(src_hbm.at[pl.ds(base, CAA)], src_c)
            pltpu.sync_copy(dst_hbm.at[pl.ds(base, CAA)], dst_c)
            pltpu.sync_copy(dstg_hbm.at[pl.ds(base, CAA)], dstg_c)
            cp0 = pltpu.async_copy(fh_hbm.at[src_c], fhr, sem0)
            cp1 = pltpu.async_copy(ft_hbm.at[dstg_c], ftr, sem1)
            cp0.wait()
            cp1.wait()

            def edge(i, _):
                tots = []
                for h in range(H):
                    acc = jnp.zeros((16,), jnp.float32)
                    for k in (2 * h, 2 * h + 1):
                        a_ = fhr[i, pl.ds(k * 16, 16)]
                        b_ = ftr[i, pl.ds(k * 16, 16)]
                        p = a_ * b_
                        p = jnp.where(p > 0, p, SLOPE * p)
                        acc = acc + p * attn_v[k]
                    tots.append(jnp.sum(acc))
                row = jnp.zeros((16,), jnp.float32)
                for h in range(H):
                    row = jnp.where(lane == h, tots[h], row)
                ebuf[i, :] = row
                return 0
            lax.fori_loop(0, CAA, edge, 0)

            for g in range(CAA // 16):
                dstv = dst_c[pl.ds(g * 16, 16)]
                plsc.addupdate_scatter(deg_l, [dstv], one16)

            pltpu.sync_copy(ebuf, e_hbm.at[pl.ds(base, CAA)])
            return 0
        lax.fori_loop(0, EPW // CAA, chunk, 0)

        pltpu.sync_copy(deg_l, deg_hbm.at[pl.ds(w * NP, NP)])

    return body(fh, ft, src_p, dst_p, dst_g, attn_flat)


def _sc_den(e_raw, dst_p, scale):

    @functools.partial(
        pl.kernel,
        out_type=jax.ShapeDtypeStruct((32, NRB, 128), jnp.float32),
        mesh=_SC_MESH,
        compiler_params=_SC_PARAMS,
        scratch_types=[
            pltpu.VMEM((CBB,), jnp.int32),
            pltpu.VMEM((CBB, 16), jnp.float32),
            pltpu.VMEM((NRB, 128), jnp.float32),
            pltpu.VMEM((NP,), jnp.float32),
        ],
    )
    def body(e_hbm, dst_hbm, scale_hbm, den_hbm,
             dst_c, ec, den_l, scale_l):
        c = lax.axis_index("c")
        s = lax.axis_index("s")
        w = s * 2 + c
        lane = lax.iota(jnp.int32, 16)
        lo = w * NRB

        pltpu.sync_copy(scale_hbm, scale_l)

        def zd(i, _):
            for kk in range(8):
                den_l[i, pl.ds(kk * 16, 16)] = jnp.zeros((16,), jnp.float32)
            return 0
        lax.fori_loop(0, NRB, zd, 0)

        def chunk(ch, _):
            base = ch * CBB
            pltpu.sync_copy(dst_hbm.at[pl.ds(base, CBB)], dst_c)
            pltpu.sync_copy(e_hbm.at[pl.ds(base, CBB)], ec)

            def grp(g, _):
                r = g * 16 + lane
                dv = dst_c[pl.ds(g * 16, 16)]
                sv = plsc.load_gather(scale_l, [dv])
                m = (dv >= lo) & (dv < lo + NRB)
                dloc = jnp.clip(dv - lo, 0, NRB - 1)
                for h in range(H):
                    hc = jnp.full((16,), h, jnp.int32)
                    ev = plsc.load_gather(ec, [r, hc])
                    ex = jnp.exp(ev * sv)
                    plsc.addupdate_scatter(den_l, [dloc, hc], ex, mask=m)
                return 0
            lax.fori_loop(0, CBB // 16, grp, 0)
            return 0
        lax.fori_loop(0, EP // CBB, chunk, 0)

        pltpu.sync_copy(den_l, den_hbm.at[w])

    return body(e_raw, dst_p, scale)


def _sc_attn(e_raw, dst_p, scale, den2):

    @functools.partial(
        pl.kernel,
        out_type=jax.ShapeDtypeStruct((2 * EP, 4), jnp.float32),
        mesh=_SC_MESH,
        compiler_params=_SC_PARAMS,
        scratch_types=[
            pltpu.VMEM((CB,), jnp.int32),
            pltpu.VMEM((CB, 16), jnp.float32),
            pltpu.VMEM((CB, 128), jnp.float32),
            pltpu.VMEM((CB, 4), jnp.float32),
            pltpu.VMEM((CB, 4), jnp.float32),
            pltpu.VMEM((NP,), jnp.float32),
            pltpu.SemaphoreType.DMA,
        ],
    )
    def body(e_hbm, dst_hbm, scale_hbm, den_hbm, a_hbm,
             dst_c, ec, denr, ab0, ab1, scale_l, sem0):
        c = lax.axis_index("c")
        s = lax.axis_index("s")
        w = s * 2 + c
        lane = lax.iota(jnp.int32, 16)

        pltpu.sync_copy(scale_hbm, scale_l)

        def chunk(ch, _):
            base = w * EPW + ch * CB
            pltpu.sync_copy(dst_hbm.at[pl.ds(base, CB)], dst_c)
            pltpu.sync_copy(e_hbm.at[pl.ds(base, CB)], ec)
            pltpu.async_copy(den_hbm.at[dst_c], denr, sem0).wait()

            def grp(g, _):
                r = g * 16 + lane
                dv = dst_c[pl.ds(g * 16, 16)]
                sv = plsc.load_gather(scale_l, [dv])
                for h in range(H):
                    hc = jnp.full((16,), h, jnp.int32)
                    ev = plsc.load_gather(ec, [r, hc])
                    ex = jnp.exp(ev * sv)
                    dnv = plsc.load_gather(denr, [r, hc])
                    av = ex / dnv
                    hc4 = jnp.full((16,), h % 4, jnp.int32)
                    if h < 4:
                        plsc.store_scatter(ab0, [r, hc4], av)
                    else:
                        plsc.store_scatter(ab1, [r, hc4], av)
                return 0
            lax.fori_loop(0, CB // 16, grp, 0)
            pltpu.sync_copy(ab0, a_hbm.at[pl.ds(base, CB)])
            pltpu.sync_copy(ab1, a_hbm.at[pl.ds(EP + base, CB)])
            return 0
        lax.fori_loop(0, EPW // CB, chunk, 0)

    return body(e_raw, dst_p, scale, den2)


def _sc_hop(curA, feT, src_p, dst_p, a2):

    @functools.partial(
        pl.kernel,
        out_type=jax.ShapeDtypeStruct((128, 2, NP), jnp.float32),
        mesh=_SC_MESH,
        compiler_params=_SC_PARAMS,
        scratch_types=[
            pltpu.VMEM((CH,), jnp.int32),
            pltpu.VMEM((CH,), jnp.int32),
            pltpu.VMEM((CH, 4), jnp.float32),
            pltpu.VMEM((2, NP), jnp.float32),
            pltpu.VMEM((2, NP), jnp.float32),
            pltpu.VMEM((2, 128), jnp.float32),
        ],
    )
    def body(cur_hbm, fet_hbm, src_hbm, dst_hbm, a_hbm, out_hbm,
             src_c, dst_c, ac, curl, agg, fel):
        c = lax.axis_index("c")
        s = lax.axis_index("s")
        w = s * 2 + c
        lane = lax.iota(jnp.int32, 16)

        for gi in range(4):
            g2 = w * 4 + gi
            c4 = g2 // 64
            lh = (g2 // 16) % 4
            pltpu.sync_copy(cur_hbm.at[g2], curl)

            for j in range(2):
                def za(i, _, j=j):
                    agg[j, pl.ds(i * 16, 16)] = jnp.zeros((16,), jnp.float32)
                    return 0
                lax.fori_loop(0, NP // 16, za, 0)

            def chunk(ch, _):
                base = ch * CH
                pltpu.sync_copy(src_hbm.at[pl.ds(base, CH)], src_c)
                pltpu.sync_copy(dst_hbm.at[pl.ds(base, CH)], dst_c)
                pltpu.sync_copy(a_hbm.at[pl.ds(c4 * EP + base, CH)], ac)

                def grp(g, _):
                    r = g * 16 + lane
                    sv = src_c[pl.ds(g * 16, 16)]
                    dv = dst_c[pl.ds(g * 16, 16)]
                    lhc = jnp.full((16,), lh, jnp.int32)
                    av = plsc.load_gather(ac, [r, lhc])
                    for j in range(2):
                        jc = jnp.full((16,), j, jnp.int32)
                        cv = plsc.load_gather(curl, [jc, sv])
                        plsc.addupdate_scatter(agg, [jc, dv], cv * av)
                    return 0
                lax.fori_loop(0, CH // 16, grp, 0)
                return 0
            lax.fori_loop(0, EP // CH, chunk, 0)

            def blch(cc, _):
                off = cc * 128
                pltpu.sync_copy(fet_hbm.at[g2, :, pl.ds(off, 128)], fel)
                for j in range(2):
                    def bl(t, _, j=j):
                        slq = pl.ds(off + t * 16, 16)
                        slf = pl.ds(t * 16, 16)
                        agg[j, slq] = ((1.0 - ALPHA) * agg[j, slq]
                                       + ALPHA * fel[j, slf])
                        return 0
                    lax.fori_loop(0, 128 // 16, bl, 0)
                return 0
            lax.fori_loop(0, NP // 128, blch, 0)
            pltpu.sync_copy(agg, out_hbm.at[g2])

    return body(curA, feT, src_p, dst_p, a2)


def _post_body(ct_ref, feat_ref, g_ref, b_ref, w1_ref, b1_ref,
               w2_ref, b2_ref, out_ref):
    cur = ct_ref[:].reshape(D, ROW_BLK).T
    rst = cur + feat_ref[:]
    mu = jnp.mean(rst, axis=-1, keepdims=True)
    var = jnp.mean((rst - mu) ** 2, axis=-1, keepdims=True)
    h2 = (rst - mu) * lax.rsqrt(var + 1e-5) * g_ref[:] + b_ref[:]
    ff = jnp.maximum(
        jnp.dot(h2, w1_ref[:], preferred_element_type=jnp.float32) + b1_ref[:],
        0.0)
    out_ref[:] = (jnp.dot(ff, w2_ref[:], preferred_element_type=jnp.float32)
                  + b2_ref[:] + rst)


def _post_stage(curT, featP, ln2_g, ln2_b, W1, b1, W2, b2):
    grid = (NP // ROW_BLK,)
    row_spec = pl.BlockSpec((ROW_BLK, D), lambda i: (i, 0))
    return pl.pallas_call(
        _post_body,
        grid=grid,
        in_specs=[pl.BlockSpec((128, 2, ROW_BLK), lambda i: (0, 0, i)),
                  row_spec,
                  pl.BlockSpec((D,), lambda i: (0,)),
                  pl.BlockSpec((D,), lambda i: (0,)),
                  pl.BlockSpec((D, FF), lambda i: (0, 0)),
                  pl.BlockSpec((FF,), lambda i: (0,)),
                  pl.BlockSpec((FF, D), lambda i: (0, 0)),
                  pl.BlockSpec((D,), lambda i: (0,))],
        out_specs=row_spec,
        out_shape=jax.ShapeDtypeStruct((NP, D), jnp.float32),
    )(curT, featP, ln2_g, ln2_b, W1, b1, W2, b2)


def kernel(feat, edge_index, W_head, W_tail, W_ent, attn, ln1_g, ln1_b,
           ln2_g, ln2_b, W1, b1, W2, b2):
    src_p = jnp.concatenate(
        [edge_index[0], jnp.zeros((EP - E,), jnp.int32)])
    dst_p = jnp.concatenate(
        [edge_index[1], jnp.full((EP - E,), N, jnp.int32)])
    dst_g = jnp.concatenate(
        [edge_index[1], jnp.zeros((EP - E,), jnp.int32)])
    attn_flat = attn.reshape(D)
    featP = jnp.concatenate(
        [feat, jnp.zeros((NP - N, D), jnp.float32)], axis=0)

    fh, ft, feT = _pre_stage(featP, W_head, W_tail, W_ent, ln1_g, ln1_b)

    e_raw, deg = _sc_edge_scores(fh, ft, src_p, dst_p, dst_g, attn_flat)
    scale = _log_stage(deg.reshape(32, NP))
    den3 = _sc_den(e_raw, dst_p, scale)
    a2 = _sc_attn(e_raw, dst_p, scale, den3.reshape(NP, 128))

    cur = feT
    for _ in range(HOPS):
        cur = _sc_hop(cur, feT, src_p, dst_p, a2)

    out = _post_stage(cur, featP, ln2_g, ln2_b, W1, b1, W2, b2)
    return out[:N]
